# baseline (speedup 1.0000x reference)
import functools

import jax
import jax.numpy as jnp
from jax import lax
from jax.experimental import pallas as pl
from jax.experimental.pallas import tpu as pltpu, tpu_sc as plsc

D_MODEL = 1024
D_SPACE = 64
KC = 8
MAX_K = 32
KEEP = 0.9
CLUSTER_SIZE = 32
TS = 256
GW = 128
CPG = GW // CLUSTER_SIZE


def _gate_pre_kernel(x_ref, wgt_ref, bgt_ref, emb_ref, ce_ref, ex_ref,
                     eg_ref, ids_ref, pmod_ref, accc_ref, *, n_groups):
    step = pl.program_id(0)

    @pl.when(step == 0)
    def _init():
        accc_ref[...] = jnp.zeros_like(accc_ref)

    x = x_ref[...]
    ht = jax.lax.dot_general(x, wgt_ref[...], (((1,), (0,)), ((), ())),
                             preferred_element_type=jnp.float32)
    ht = ht + bgt_ref[...]
    h = ht[:, :D_SPACE] * (1.0 / KEEP)
    tau = ht[:, D_SPACE:D_SPACE + 1]

    ce = ce_ref[...]
    inv_c = 1.0 / (jnp.sqrt(jnp.sum(ce * ce, axis=0, keepdims=True)) + 1e-08)
    cs = jax.lax.dot_general(h, ce * inv_c, (((1,), (0,)), ((), ())),
                             preferred_element_type=jnp.float32)
    n_clusters = cs.shape[-1]

    cmax = jnp.max(cs, axis=-1, keepdims=True)
    ce_exp = jnp.exp(cs - cmax)
    probs = ce_exp / jnp.sum(ce_exp, axis=-1, keepdims=True)
    accc_ref[...] += jnp.broadcast_to(
        jnp.sum(probs, axis=0, keepdims=True), accc_ref.shape)

    lane = jax.lax.broadcasted_iota(jnp.int32, cs.shape, 1)
    w = cs
    ids = []
    act_b = None
    for _ in range(KC):
        m = jnp.max(w, axis=-1, keepdims=True)
        idx = jnp.min(jnp.where(w == m, lane, n_clusters), axis=-1,
                      keepdims=True)
        ids.append(idx)
        hit = lane == idx
        act_b = hit if act_b is None else (act_b | hit)
        w = jnp.where(hit, -1e30, w)
    ids8 = jnp.concatenate(ids, axis=1)
    act = act_b.astype(jnp.float32)

    tok = (step * TS
           + jax.lax.broadcasted_iota(jnp.int32, (TS, 1), 0))
    ids_ref[...] = tok * n_groups + (ids8 >> 2)
    pmod_ref[...] = ids8 & (CPG - 1)

    act_n = jax.lax.dot_general(act, ex_ref[...], (((1,), (0,)), ((), ())),
                                preferred_element_type=jnp.float32) > 0.5

    emb = emb_ref[...]
    inv_n = 1.0 / (jnp.sqrt(jnp.sum(emb * emb, axis=0, keepdims=True)) + 1e-08)
    scores = jax.lax.dot_general(h, emb * inv_n, (((1,), (0,)), ((), ())),
                                 preferred_element_type=jnp.float32)

    raw = scores - tau
    eg_ref[...] = jnp.where(act_n, jnp.maximum(jnp.exp(raw) - 1.0, 0.0), 0.0)


@functools.lru_cache(maxsize=None)
def _make_sc_gather(n_rows, n_idx):
    info = plsc.get_sparse_core_info()
    nw = info.num_cores * info.num_subcores
    ch = 128
    b_per_w = n_idx // nw
    assert b_per_w % ch == 0
    mesh = plsc.VectorSubcoreMesh(core_axis_name="c", subcore_axis_name="s")

    @functools.partial(
        pl.kernel, mesh=mesh,
        out_type=jax.ShapeDtypeStruct((n_idx, GW), jnp.float32),
        scratch_types=[
            pltpu.VMEM((ch,), jnp.int32),
            pltpu.VMEM((ch, GW), jnp.float32),
            pltpu.SemaphoreType.DMA,
        ],
    )
    def sc_gather(table_hbm, idx_hbm, out_hbm, idx_v, rows_v, sem):
        wid = lax.axis_index("s") * info.num_cores + lax.axis_index("c")

        @pl.loop(0, b_per_w // ch)
        def _(j):
            base = wid * b_per_w + j * ch
            pltpu.sync_copy(idx_hbm.at[pl.ds(base, ch)], idx_v)
            pltpu.async_copy(table_hbm.at[idx_v], rows_v, sem).wait()
            pltpu.sync_copy(rows_v, out_hbm.at[pl.ds(base, ch)])

    return sc_gather


def _gate_post_kernel(eg_ref, cp_ref, pmod_ref, out_ref, accn_ref):
    step = pl.program_id(0)

    @pl.when(step == 0)
    def _init():
        accn_ref[...] = jnp.zeros_like(accn_ref)

    pmod = pmod_ref[...]
    grp_lane = jax.lax.broadcasted_iota(
        jnp.int32, (TS, GW), 1) >> 5
    pieces = []
    for k in range(KC):
        sel = grp_lane == pmod[:, k:k + 1]
        pieces.append(jnp.where(sel, cp_ref[:, k * GW:(k + 1) * GW], 0.0))
    masked = jnp.concatenate(pieces, axis=1)

    m1 = jnp.max(masked, axis=-1, keepdims=True)

    w_bits = jax.lax.bitcast_convert_type(masked, jnp.int32)
    lo = jnp.zeros_like(m1, dtype=jnp.int32)
    hi = jax.lax.bitcast_convert_type(m1, jnp.int32) + 1
    for _ in range(31):
        mid = lo + ((hi - lo) >> 1)
        cnt = jnp.sum((w_bits >= mid).astype(jnp.float32), axis=-1,
                      keepdims=True)
        pred = cnt >= MAX_K
        lo = jnp.where(pred, mid, lo)
        hi = jnp.where(pred, hi, mid)

    kept_c = jnp.where(w_bits >= lo, masked, 0.0)
    gsum = jnp.sum(kept_c, axis=-1, keepdims=True) + 1e-08
    scale = jnp.tanh(m1) / gsum

    eg = eg_ref[...]
    eg_bits = jax.lax.bitcast_convert_type(eg, jnp.int32)
    out = jnp.where(eg_bits >= lo, eg, 0.0) * scale
    out_ref[...] = out
    accn_ref[...] += jnp.broadcast_to(
        jnp.sum(out, axis=0, keepdims=True), accn_ref.shape)


def _run_gate_pre(x2d, wgt, bgt, emb_t, ce_t, ex, n_clusters, n_neurons):
    tokens = x2d.shape[0]
    grid = tokens // TS
    n_groups = n_neurons // GW
    pre = functools.partial(_gate_pre_kernel, n_groups=n_groups)
    eg, ids, pmod, accc = pl.pallas_call(
        pre,
        grid=(grid,),
        in_specs=[
            pl.BlockSpec((TS, D_MODEL), lambda i: (i, 0)),
            pl.BlockSpec((D_MODEL, 128), lambda i: (0, 0)),
            pl.BlockSpec((1, 128), lambda i: (0, 0)),
            pl.BlockSpec((D_SPACE, n_neurons), lambda i: (0, 0)),
            pl.BlockSpec((D_SPACE, n_clusters), lambda i: (0, 0)),
            pl.BlockSpec((n_clusters, n_neurons), lambda i: (0, 0)),
        ],
        out_specs=[
            pl.BlockSpec((TS, n_neurons), lambda i: (i, 0)),
            pl.BlockSpec((TS, KC), lambda i: (i, 0)),
            pl.BlockSpec((TS, KC), lambda i: (i, 0)),
            pl.BlockSpec((8, n_clusters), lambda i: (0, 0)),
        ],
        out_shape=[
            jax.ShapeDtypeStruct((tokens, n_neurons), jnp.float32),
            jax.ShapeDtypeStruct((tokens, KC), jnp.int32),
            jax.ShapeDtypeStruct((tokens, KC), jnp.int32),
            jax.ShapeDtypeStruct((8, n_clusters), jnp.float32),
        ],
    )(x2d, wgt, bgt, emb_t, ce_t, ex)
    return eg, ids, pmod, accc


def _run_gate_gather(eg, ids, n_neurons):
    tokens = eg.shape[0]
    n_groups = n_neurons // GW
    n_idx = tokens * KC
    gather = _make_sc_gather(tokens * n_groups, n_idx)
    compact = gather(eg.reshape(tokens * n_groups, GW),
                     ids.reshape(n_idx))
    return compact.reshape(tokens, KC * GW)


def _run_gate_post(eg, compact, pmod, n_neurons):
    tokens = eg.shape[0]
    grid = tokens // TS
    out, accn = pl.pallas_call(
        _gate_post_kernel,
        grid=(grid,),
        in_specs=[
            pl.BlockSpec((TS, n_neurons), lambda i: (i, 0)),
            pl.BlockSpec((TS, KC * GW), lambda i: (i, 0)),
            pl.BlockSpec((TS, KC), lambda i: (i, 0)),
        ],
        out_specs=[
            pl.BlockSpec((TS, n_neurons), lambda i: (i, 0)),
            pl.BlockSpec((8, n_neurons), lambda i: (0, 0)),
        ],
        out_shape=[
            jax.ShapeDtypeStruct((tokens, n_neurons), jnp.float32),
            jax.ShapeDtypeStruct((8, n_neurons), jnp.float32),
        ],
    )(eg, compact, pmod)
    return out, accn[0]


def _aux(freq_sum, tokens, n):
    freq = freq_sum / tokens
    return ((freq - 1.0 / n) ** 2).sum() * n


def kernel(x, neuron_emb, W_attn, b_attn, W_know, b_know, W_tau_attn,
           b_tau_attn, W_tau_know, b_tau_know, cluster_emb_qk, cluster_emb_v,
           cluster_emb_know):
    B, S, _ = x.shape
    tokens = B * S
    x2d = x.reshape(tokens, D_MODEL)

    n_qk = cluster_emb_qk.shape[0] * CLUSTER_SIZE
    n_v = cluster_emb_v.shape[0] * CLUSTER_SIZE
    n_know = cluster_emb_know.shape[0] * CLUSTER_SIZE

    qk_emb_t = neuron_emb[:n_qk].T
    v_emb_t = neuron_emb[n_qk:n_qk + n_v].T
    know_emb_t = neuron_emb[n_qk + n_v:].T

    def mk_wgt(w_h, w_tau_col):
        pad = jnp.zeros((D_MODEL, 128 - D_SPACE - 1), jnp.float32)
        return jnp.concatenate([w_h, w_tau_col, pad], axis=1)

    def mk_bgt(b_h, b_tau_col):
        pad = jnp.zeros((128 - D_SPACE - 1,), jnp.float32)
        return jnp.concatenate([b_h, b_tau_col, pad])[None, :]

    def mk_ex(n_clusters):
        n = n_clusters * CLUSTER_SIZE
        rows = jnp.arange(n_clusters)[:, None]
        cols = jnp.arange(n)[None, :] // CLUSTER_SIZE
        return (rows == cols).astype(jnp.float32)

    specs = [
        (W_attn[:, 0:D_SPACE], b_attn[0:D_SPACE], W_tau_attn[:, 0:1],
         b_tau_attn[0:1], qk_emb_t, cluster_emb_qk.T, n_qk),
        (W_attn[:, D_SPACE:2 * D_SPACE], b_attn[D_SPACE:2 * D_SPACE],
         W_tau_attn[:, 1:2], b_tau_attn[1:2], qk_emb_t, cluster_emb_qk.T,
         n_qk),
        (W_attn[:, 2 * D_SPACE:], b_attn[2 * D_SPACE:], W_tau_attn[:, 2:3],
         b_tau_attn[2:3], v_emb_t, cluster_emb_v.T, n_v),
        (W_know, b_know, W_tau_know, b_tau_know, know_emb_t,
         cluster_emb_know.T, n_know),
    ]
    pres = []
    for w_h, b_h, w_t, b_t, emb_t, ce_t, n in specs:
        c = n // CLUSTER_SIZE
        pres.append(_run_gate_pre(
            x2d, mk_wgt(w_h, w_t), mk_bgt(b_h, b_t), emb_t, ce_t, mk_ex(c),
            c, n))
    compacts = [_run_gate_gather(pres[i][0], pres[i][1], specs[i][6])
                for i in range(4)]
    gates = []
    auxs = []
    for i in range(4):
        eg, _, pmod, accc = pres[i]
        n = specs[i][6]
        out, accn = _run_gate_post(eg, compacts[i], pmod, n)
        gates.append(out.reshape(B, S, n))
        auxs.append(_aux(accc[0], tokens, n // CLUSTER_SIZE)
                    + _aux(accn, tokens, n))

    aux = auxs[0] + auxs[1] + auxs[2] + auxs[3]
    return gates[0], gates[1], gates[2], gates[3], aux


# speedup vs baseline: 1.1774x; 1.1774x over previous
import functools

import jax
import jax.numpy as jnp
from jax.experimental import pallas as pl

D_MODEL = 1024
D_SPACE = 64
KC = 8
MAX_K = 32
KEEP = 0.9
CLUSTER_SIZE = 32
TS = 256


def _gate_block_kernel(x_ref, wgt_ref, bgt_ref, emb_ref, ce_ref, ex_ref,
                       out_ref, accc_ref, accn_ref, *, n_clusters, n_neurons):
    step = pl.program_id(0)

    @pl.when(step == 0)
    def _init():
        accc_ref[...] = jnp.zeros_like(accc_ref)
        accn_ref[...] = jnp.zeros_like(accn_ref)

    x = x_ref[...]
    ht = jax.lax.dot_general(x, wgt_ref[...], (((1,), (0,)), ((), ())),
                             preferred_element_type=jnp.float32)
    ht = ht + bgt_ref[...]
    h = ht[:, :D_SPACE] * (1.0 / KEEP)
    tau = ht[:, D_SPACE:D_SPACE + 1]

    ce = ce_ref[...]
    inv_c = 1.0 / (jnp.sqrt(jnp.sum(ce * ce, axis=0, keepdims=True)) + 1e-08)
    cs = jax.lax.dot_general(h, ce * inv_c, (((1,), (0,)), ((), ())),
                             preferred_element_type=jnp.float32)

    cmax = jnp.max(cs, axis=-1, keepdims=True)
    ce_exp = jnp.exp(cs - cmax)
    probs = ce_exp / jnp.sum(ce_exp, axis=-1, keepdims=True)
    accc_ref[...] += jnp.broadcast_to(
        jnp.sum(probs, axis=0, keepdims=True), accc_ref.shape)

    w = cs
    t8 = None
    for _ in range(KC):
        t8 = jnp.max(w, axis=-1, keepdims=True)
        w = jnp.where(w == t8, -1e30, w)
    act = (cs >= t8).astype(jnp.float32)

    act_n = jax.lax.dot_general(act, ex_ref[...], (((1,), (0,)), ((), ())),
                                preferred_element_type=jnp.float32) > 0.5

    emb = emb_ref[...]
    inv_n = 1.0 / (jnp.sqrt(jnp.sum(emb * emb, axis=0, keepdims=True)) + 1e-08)
    scores = jax.lax.dot_general(h, emb * inv_n, (((1,), (0,)), ((), ())),
                                 preferred_element_type=jnp.float32)

    raw = scores - tau
    eg = jnp.where(act_n, jnp.maximum(jnp.exp(raw) - 1.0, 0.0), 0.0)

    m1 = jnp.max(eg, axis=-1, keepdims=True)

    w_bits = jax.lax.bitcast_convert_type(eg, jnp.int32)
    lo = jnp.zeros_like(m1, dtype=jnp.int32)
    hi = jax.lax.bitcast_convert_type(m1, jnp.int32) + 1
    for _ in range(31):
        mid = lo + ((hi - lo) >> 1)
        cnt = jnp.sum((w_bits >= mid).astype(jnp.float32), axis=-1,
                      keepdims=True)
        pred = cnt >= MAX_K
        lo = jnp.where(pred, mid, lo)
        hi = jnp.where(pred, hi, mid)

    kept = jnp.where(w_bits >= lo, eg, 0.0)
    gsum = jnp.sum(kept, axis=-1, keepdims=True) + 1e-08
    out = kept * (jnp.tanh(m1) / gsum)
    out_ref[...] = out
    accn_ref[...] += jnp.broadcast_to(
        jnp.sum(out, axis=0, keepdims=True), accn_ref.shape)


def _run_gate(x2d, wgt, bgt, emb_t, ce_t, ex, n_clusters, n_neurons):
    tokens = x2d.shape[0]
    grid = tokens // TS
    kern = functools.partial(_gate_block_kernel, n_clusters=n_clusters,
                             n_neurons=n_neurons)
    out, accc, accn = pl.pallas_call(
        kern,
        grid=(grid,),
        in_specs=[
            pl.BlockSpec((TS, D_MODEL), lambda i: (i, 0)),
            pl.BlockSpec((D_MODEL, 128), lambda i: (0, 0)),
            pl.BlockSpec((1, 128), lambda i: (0, 0)),
            pl.BlockSpec((D_SPACE, n_neurons), lambda i: (0, 0)),
            pl.BlockSpec((D_SPACE, n_clusters), lambda i: (0, 0)),
            pl.BlockSpec((n_clusters, n_neurons), lambda i: (0, 0)),
        ],
        out_specs=[
            pl.BlockSpec((TS, n_neurons), lambda i: (i, 0)),
            pl.BlockSpec((8, n_clusters), lambda i: (0, 0)),
            pl.BlockSpec((8, n_neurons), lambda i: (0, 0)),
        ],
        out_shape=[
            jax.ShapeDtypeStruct((tokens, n_neurons), jnp.float32),
            jax.ShapeDtypeStruct((8, n_clusters), jnp.float32),
            jax.ShapeDtypeStruct((8, n_neurons), jnp.float32),
        ],
    )(x2d, wgt, bgt, emb_t, ce_t, ex)
    return out, accc[0], accn[0]


def _aux(freq_sum, tokens, n):
    freq = freq_sum / tokens
    return ((freq - 1.0 / n) ** 2).sum() * n


def kernel(x, neuron_emb, W_attn, b_attn, W_know, b_know, W_tau_attn,
           b_tau_attn, W_tau_know, b_tau_know, cluster_emb_qk, cluster_emb_v,
           cluster_emb_know):
    B, S, _ = x.shape
    tokens = B * S
    x2d = x.reshape(tokens, D_MODEL)

    n_qk = cluster_emb_qk.shape[0] * CLUSTER_SIZE
    n_v = cluster_emb_v.shape[0] * CLUSTER_SIZE
    n_know = cluster_emb_know.shape[0] * CLUSTER_SIZE

    qk_emb_t = neuron_emb[:n_qk].T
    v_emb_t = neuron_emb[n_qk:n_qk + n_v].T
    know_emb_t = neuron_emb[n_qk + n_v:].T

    def mk_wgt(w_h, w_tau_col):
        pad = jnp.zeros((D_MODEL, 128 - D_SPACE - 1), jnp.float32)
        return jnp.concatenate([w_h, w_tau_col, pad], axis=1)

    def mk_bgt(b_h, b_tau_col):
        pad = jnp.zeros((128 - D_SPACE - 1,), jnp.float32)
        return jnp.concatenate([b_h, b_tau_col, pad])[None, :]

    def mk_ex(n_clusters):
        n = n_clusters * CLUSTER_SIZE
        rows = jnp.arange(n_clusters)[:, None]
        cols = jnp.arange(n)[None, :] // CLUSTER_SIZE
        return (rows == cols).astype(jnp.float32)

    gates = []
    auxs = []
    specs = [
        (W_attn[:, 0:D_SPACE], b_attn[0:D_SPACE], W_tau_attn[:, 0:1],
         b_tau_attn[0:1], qk_emb_t, cluster_emb_qk.T, n_qk),
        (W_attn[:, D_SPACE:2 * D_SPACE], b_attn[D_SPACE:2 * D_SPACE],
         W_tau_attn[:, 1:2], b_tau_attn[1:2], qk_emb_t, cluster_emb_qk.T,
         n_qk),
        (W_attn[:, 2 * D_SPACE:], b_attn[2 * D_SPACE:], W_tau_attn[:, 2:3],
         b_tau_attn[2:3], v_emb_t, cluster_emb_v.T, n_v),
        (W_know, b_know, W_tau_know, b_tau_know, know_emb_t,
         cluster_emb_know.T, n_know),
    ]
    for w_h, b_h, w_t, b_t, emb_t, ce_t, n in specs:
        c = n // CLUSTER_SIZE
        out, accc, accn = _run_gate(
            x2d, mk_wgt(w_h, w_t), mk_bgt(b_h, b_t), emb_t, ce_t, mk_ex(c),
            c, n)
        gates.append(out.reshape(B, S, n))
        auxs.append(_aux(accc, tokens, c) + _aux(accn, tokens, n))

    aux = auxs[0] + auxs[1] + auxs[2] + auxs[3]
    return gates[0], gates[1], gates[2], gates[3], aux
